# baseline (speedup 1.0000x reference)
import jax
import jax.numpy as jnp
from jax import lax
from jax.experimental import pallas as pl
from jax.experimental.pallas import tpu as pltpu
from jax.experimental.pallas import tpu_sc as plsc

ROWS = 128
COLS = 32768
SC_COLS = 8192
TC_COLS = COLS - SC_COLS

BW = 2048
TC_GRID = TC_COLS // BW


def _tc_body(x_ref, o_ref):
    v = x_ref[...]
    ridx = lax.broadcasted_iota(jnp.int32, (ROWS, BW), 0)
    mx = jnp.max(v, axis=0, keepdims=True)
    cand = jnp.where(v == mx, ridx, jnp.int32(ROWS))
    o_ref[...] = jnp.min(cand, axis=0, keepdims=True)


NC = 2
NS = 16
L = 16
NW = NC * NS
CPW = SC_COLS // NW
CHUNK = 128
NCHUNK = CPW // CHUNK
G = CHUNK // L


def _sc_body(x_hbm, out_hbm, buf0, buf1, idx_v, sem0, sem1):
    wid = lax.axis_index("s") * NC + lax.axis_index("c")
    base = wid * CPW
    bufs = (buf0, buf1)
    sems = (sem0, sem1)

    def src(ci):
        return x_hbm.at[:, pl.ds(TC_COLS + base + ci * CHUNK, CHUNK)]

    copies = [None] * NCHUNK
    copies[0] = pltpu.async_copy(src(0), bufs[0], sems[0])
    for ci in range(NCHUNK):
        if ci + 1 < NCHUNK:
            copies[ci + 1] = pltpu.async_copy(
                src(ci + 1), bufs[(ci + 1) % 2], sems[(ci + 1) % 2])
        copies[ci].wait()
        buf = bufs[ci % 2]

        maxv0 = tuple(buf[0, pl.ds(g * L, L)] for g in range(G))
        maxi0 = tuple(jnp.zeros((L,), jnp.int32) for _ in range(G))

        def row_step(r, carry, buf=buf):
            mv, mi = carry
            ridx = jnp.full((L,), r, jnp.int32)
            nmv, nmi = [], []
            for g in range(G):
                v = buf[r, pl.ds(g * L, L)]
                gt = v > mv[g]
                nmv.append(jnp.where(gt, v, mv[g]))
                nmi.append(jnp.where(gt, ridx, mi[g]))
            return tuple(nmv), tuple(nmi)

        _, maxi = lax.fori_loop(1, ROWS, row_step, (maxv0, maxi0))
        for g in range(G):
            idx_v[pl.ds(ci * CHUNK + g * L, L)] = maxi[g]

    pltpu.sync_copy(idx_v, out_hbm.at[pl.ds(base, CPW)])


@jax.jit
def _argmax_hybrid(x):
    mesh = plsc.VectorSubcoreMesh(core_axis_name="c", subcore_axis_name="s")
    sc_f = pl.kernel(
        _sc_body,
        out_type=jax.ShapeDtypeStruct((SC_COLS,), jnp.int32),
        mesh=mesh,
        scratch_types=[
            pltpu.VMEM((ROWS, CHUNK), jnp.float32),
            pltpu.VMEM((ROWS, CHUNK), jnp.float32),
            pltpu.VMEM((CPW,), jnp.int32),
            pltpu.SemaphoreType.DMA,
            pltpu.SemaphoreType.DMA,
        ],
    )
    sc_out = sc_f(x)
    tc_out = pl.pallas_call(
        _tc_body,
        grid=(TC_GRID,),
        in_specs=[pl.BlockSpec((ROWS, BW), lambda i: (0, i))],
        out_specs=pl.BlockSpec((1, BW), lambda i: (0, i)),
        out_shape=jax.ShapeDtypeStruct((1, TC_COLS), jnp.int32),
    )(x)
    return jnp.concatenate([tc_out, sc_out.reshape(1, SC_COLS)], axis=1)


def kernel(x):
    return _argmax_hybrid(x).astype(jnp.int64)


# speedup vs baseline: 1.4396x; 1.4396x over previous
import jax
import jax.numpy as jnp
from jax import lax
from jax.experimental import pallas as pl
from jax.experimental.pallas import tpu as pltpu

ROWS = 128
COLS = 32768
BR = 8
GRID = ROWS // BR


def _tc_body(x_ref, o_ref, mv_ref):
    i = pl.program_id(0)
    v = x_ref[...]
    ridx = lax.broadcasted_iota(jnp.int32, (BR, COLS), 0) + i * BR
    bmx = jnp.max(v, axis=0, keepdims=True)
    bidx = jnp.min(jnp.where(v == bmx, ridx, jnp.int32(ROWS)),
                   axis=0, keepdims=True)

    @pl.when(i == 0)
    def _():
        mv_ref[...] = bmx
        o_ref[...] = bidx

    @pl.when(i > 0)
    def _():
        upd = bmx > mv_ref[...]
        mv_ref[...] = jnp.where(upd, bmx, mv_ref[...])
        o_ref[...] = jnp.where(upd, bidx, o_ref[...])


@jax.jit
def _argmax_tc(x):
    return pl.pallas_call(
        _tc_body,
        grid=(GRID,),
        in_specs=[pl.BlockSpec((BR, COLS), lambda i: (i, 0))],
        out_specs=pl.BlockSpec((1, COLS), lambda i: (0, 0)),
        out_shape=jax.ShapeDtypeStruct((1, COLS), jnp.int32),
        scratch_shapes=[pltpu.VMEM((1, COLS), jnp.float32)],
    )(x)


def kernel(x):
    return _argmax_tc(x).astype(jnp.int64)


# speedup vs baseline: 2.8526x; 1.9815x over previous
import jax
import jax.numpy as jnp
from jax import lax
from jax.experimental import pallas as pl
from jax.experimental.pallas import tpu as pltpu

ROWS = 128
COLS = 32768
BW = 4096
GRID = COLS // BW


def _tc_body(x_ref, o_ref):
    v = x_ref[...]
    ridx = lax.broadcasted_iota(jnp.int32, (ROWS, BW), 0)
    mx = jnp.max(v, axis=0, keepdims=True)
    cand = jnp.where(v == mx, ridx, jnp.int32(ROWS))
    o_ref[...] = jnp.min(cand, axis=0, keepdims=True)


@jax.jit
def _argmax_tc(x):
    return pl.pallas_call(
        _tc_body,
        grid=(GRID,),
        in_specs=[pl.BlockSpec((ROWS, BW), lambda i: (0, i))],
        out_specs=pl.BlockSpec((1, BW), lambda i: (0, i)),
        out_shape=jax.ShapeDtypeStruct((1, COLS), jnp.int32),
    )(x)


def kernel(x):
    return _argmax_tc(x).astype(jnp.int64)


# speedup vs baseline: 3.4485x; 1.2089x over previous
import jax
import jax.numpy as jnp
from jax import lax
from jax.experimental import pallas as pl
from jax.experimental.pallas import tpu as pltpu

ROWS = 128
COLS = 32768
BW = 8192
GRID = COLS // BW


def _tc_body(x_ref, o_ref):
    v = x_ref[...]
    ridx = lax.broadcasted_iota(jnp.int32, (ROWS, BW), 0)
    mx = jnp.max(v, axis=0, keepdims=True)
    cand = jnp.where(v == mx, ridx, jnp.int32(ROWS))
    o_ref[...] = jnp.min(cand, axis=0, keepdims=True)


@jax.jit
def _argmax_tc(x):
    return pl.pallas_call(
        _tc_body,
        grid=(GRID,),
        in_specs=[pl.BlockSpec((ROWS, BW), lambda i: (0, i))],
        out_specs=pl.BlockSpec((1, BW), lambda i: (0, i)),
        out_shape=jax.ShapeDtypeStruct((1, COLS), jnp.int32),
    )(x)


def kernel(x):
    return _argmax_tc(x).astype(jnp.int64)
